# baseline (speedup 1.0000x reference)
import functools
import math

import jax
import jax.numpy as jnp
from jax import lax
from jax.experimental import pallas as pl
from jax.experimental.pallas import tpu as pltpu
from jax.experimental.pallas import tpu_sc as plsc

_M = 0.5
_S = 64.0
_COS_M = math.cos(_M)
_SIN_M = math.sin(_M)
_THRESHOLD = math.cos(math.pi - _M)
_MM = math.sin(math.pi - _M) * _M

_B = 1024
_V = 100000
_RB = 16
_CH = 512

_NC = 2
_NS = 16
_L = 16
_NW = _NC * _NS
_BPW = _B // _NW


def _sc_gather_body(logits_hbm, labels_hbm, out_hbm, lab_v, win_v, chunk_v, sem):
    wid = lax.axis_index("s") * _NC + lax.axis_index("c")
    base = wid * _BPW
    pltpu.sync_copy(labels_hbm.at[pl.ds(base, _BPW)], lab_v)
    copies = []
    for g in range(_BPW // _L):
        labs = lab_v[pl.ds(g * _L, _L)]
        for l in range(_L):
            j = g * _L + l
            col0 = pl.multiple_of((labs[l] >> 7) << 7, 128)
            row0 = pl.multiple_of(base + (j & ~7), 8)
            cp = pltpu.make_async_copy(
                logits_hbm.at[pl.ds(row0, 8), pl.ds(col0, 128)],
                win_v.at[j],
                sem,
            )
            cp.start()
            copies.append(cp)
    for cp in copies:
        cp.wait()
    for g in range(_BPW // _L):
        labs = lab_v[pl.ds(g * _L, _L)]
        for l in range(_L):
            j = g * _L + l
            rel0 = pl.multiple_of(((labs[l] & 127) >> 4) << 4, 16)
            chunk_v[j, :] = win_v[j, j & 7, pl.ds(rel0, _L)]
    pltpu.sync_copy(chunk_v, out_hbm.at[pl.ds(base, _BPW)])


@functools.cache
def _sc_gather():
    return functools.partial(
        pl.kernel,
        out_type=jax.ShapeDtypeStruct((_B, _L), jnp.float32),
        mesh=plsc.VectorSubcoreMesh(
            core_axis_name="c", subcore_axis_name="s", num_cores=_NC
        ),
        scratch_types=[
            pltpu.VMEM((_BPW,), jnp.int32),
            pltpu.VMEM((_BPW, 8, 128), jnp.float32),
            pltpu.VMEM((_BPW, _L), jnp.float32),
            pltpu.SemaphoreType.DMA,
        ],
    )(_sc_gather_body)


_NBLK = _B // _RB


def _dense_body(
    chunk_ref, lab_all_ref, x_hbm, o_hbm, in_buf, out_buf, sem_in, sem_out
):
    s = pl.program_id(0)

    @pl.when(s < _NBLK)
    def _start_in():
        slot = s % 2
        pltpu.make_async_copy(
            x_hbm.at[pl.ds(s * _RB, _RB), :], in_buf.at[slot], sem_in.at[slot]
        ).start()

    @pl.when((s >= 1) & (s <= _NBLK))
    def _compute():
        b = s - 1
        slot = b % 2
        row0 = b * _RB
        pltpu.make_async_copy(
            x_hbm.at[pl.ds(row0, _RB), :], in_buf.at[slot], sem_in.at[slot]
        ).wait()

        @pl.when(b >= 2)
        def _wait_out_slot():
            pltpu.make_async_copy(
                out_buf.at[slot],
                o_hbm.at[pl.ds((b - 2) * _RB, _RB), :],
                sem_out.at[slot],
            ).wait()

        lab_all = lab_all_ref[...]
        lanes = lax.broadcasted_iota(jnp.int32, (_B, _L), 1)
        sel = jnp.where(lanes == (lab_all & (_L - 1)), chunk_ref[...], 0.0)
        tgt_all = jnp.clip(jnp.sum(sel, axis=1, keepdims=True), -1.0, 1.0)
        t = jnp.mean(tgt_all) * 0.01
        chunk_rb = chunk_ref[pl.ds(row0, _RB), :]
        lab_rb = lab_all_ref[pl.ds(row0, _RB), :]
        lanes_rb = lax.broadcasted_iota(jnp.int32, (_RB, _L), 1)
        sel_rb = jnp.where(lanes_rb == (lab_rb & (_L - 1)), chunk_rb, 0.0)
        tgt = jnp.clip(jnp.sum(sel_rb, axis=1, keepdims=True), -1.0, 1.0)
        sin_t = jnp.sqrt(1.0 - tgt * tgt)
        ctm = tgt * _COS_M - sin_t * _SIN_M
        ftl = jnp.where(tgt > _THRESHOLD, ctm, tgt - _MM)
        for c0 in range(0, _V, _CH):
            w = min(_CH, _V - c0)
            c = jnp.clip(in_buf[slot, :, c0 : c0 + w], -1.0, 1.0)
            out = jnp.where(c > ctm, c * (t + c), c)
            cols = lax.broadcasted_iota(jnp.int32, (_RB, w), 1) + c0
            out = jnp.where(cols == lab_rb, ftl, out)
            out_buf[slot, :, c0 : c0 + w] = out * _S
        pltpu.make_async_copy(
            out_buf.at[slot], o_hbm.at[pl.ds(row0, _RB), :], sem_out.at[slot]
        ).start()

    @pl.when(s == _NBLK + 1)
    def _drain():
        pltpu.make_async_copy(
            out_buf.at[0],
            o_hbm.at[pl.ds((_NBLK - 2) * _RB, _RB), :],
            sem_out.at[0],
        ).wait()
        pltpu.make_async_copy(
            out_buf.at[1],
            o_hbm.at[pl.ds((_NBLK - 1) * _RB, _RB), :],
            sem_out.at[1],
        ).wait()


def kernel(logits, labels):
    labels = labels.astype(jnp.int32)
    chunks = _sc_gather()(logits, labels)
    dense = pl.pallas_call(
        _dense_body,
        grid=(_NBLK + 2,),
        in_specs=[
            pl.BlockSpec((_B, _L), lambda i: (0, 0)),
            pl.BlockSpec((_B, 1), lambda i: (0, 0)),
            pl.BlockSpec(memory_space=pl.ANY),
        ],
        out_specs=pl.BlockSpec(memory_space=pl.ANY),
        out_shape=jax.ShapeDtypeStruct((_B, _V), jnp.float32),
        scratch_shapes=[
            pltpu.VMEM((2, _RB, _V), jnp.float32),
            pltpu.VMEM((2, _RB, _V), jnp.float32),
            pltpu.SemaphoreType.DMA((2,)),
            pltpu.SemaphoreType.DMA((2,)),
        ],
    )
    return dense(chunks, labels.reshape(_B, 1), logits)
